# Initial kernel scaffold; baseline (speedup 1.0000x reference)
import functools

import jax
import jax.numpy as jnp
from jax import lax
from jax.experimental import pallas as pl
from jax.experimental.pallas import tpu as pltpu
from jax.experimental.pallas import tpu_sc as plsc

NC = 2
NS = 16
NW = NC * NS
L = 16

N = 10000
R = 8
H = 128
NB = 64
STEPS = 3

CH = 4
SEGP = 80128
SEG_T = SEGP // NS
N_ACC = N + 16
PAD_DST = N + 8


def _cdiv(a, b):
    return (a + b - 1) // b


def _mesh():
    return plsc.VectorSubcoreMesh(
        core_axis_name="c", subcore_axis_name="s", num_cores=NC, num_subcores=NS)


def _fill_zeros(ref, nrows):
    zero = jnp.zeros((L,), jnp.float32)

    @pl.loop(0, nrows)
    def _row(r):
        for j in range(128 // L):
            ref[r, pl.ds(j * L, L)] = zero


def _make_sc_cnt(er_rows, rows_per_w):
    n_chunks = rows_per_w // CH

    @functools.partial(
        pl.kernel,
        out_type=jax.ShapeDtypeStruct((NC * SEGP,), jnp.float32),
        mesh=_mesh(),
        scratch_types=[
            pltpu.VMEM((CH, 128), jnp.int32),
            pltpu.VMEM((CH, 128), jnp.int32),
            pltpu.VMEM((CH, 128), jnp.int32),
            pltpu.VMEM((128,), jnp.float32),
            pltpu.VMEM((SEG_T,), jnp.float32),
            pltpu.VMEM_SHARED((SEGP,), jnp.float32),
        ],
    )
    def sc_cnt(dst_hbm, et_hbm, out_hbm, dst_v, et_v, seg_v, ones_v, zer_v, cnt_sh):
        c = lax.axis_index("c")
        s = lax.axis_index("s")
        wid = c * NS + s

        zero = jnp.zeros((L,), jnp.float32)
        one = jnp.ones((L,), jnp.float32)

        @pl.loop(0, SEG_T // L)
        def _z(i):
            zer_v[pl.ds(i * L, L)] = zero

        for j in range(128 // L):
            ones_v[pl.ds(j * L, L)] = one

        pltpu.sync_copy(zer_v, cnt_sh.at[pl.ds(s * SEG_T, SEG_T)])
        plsc.subcore_barrier()

        row0 = wid * rows_per_w

        @pl.loop(0, n_chunks)
        def _chunk(k):
            base = row0 + k * CH
            pltpu.sync_copy(dst_hbm.at[pl.ds(base, CH)], dst_v)
            pltpu.sync_copy(et_hbm.at[pl.ds(base, CH)], et_v)
            for jr in range(CH):
                for jj in range(128 // L):
                    sl = pl.ds(jj * L, L)
                    seg_v[jr, sl] = dst_v[jr, sl] * R + et_v[jr, sl]
            for jr in range(CH):
                pltpu.sync_copy(ones_v, cnt_sh.at[seg_v.at[jr]], add=True)

        plsc.subcore_barrier()
        pltpu.sync_copy(cnt_sh.at[pl.ds(s * SEG_T, SEG_T)],
                        out_hbm.at[pl.ds(c * SEGP + s * SEG_T, SEG_T)])

    return sc_cnt


def _make_sc_wgt(er_rows, rows_per_w):
    n_chunks = rows_per_w // CH

    @functools.partial(
        pl.kernel,
        out_type=(jax.ShapeDtypeStruct((er_rows, 128), jnp.float32),
                  jax.ShapeDtypeStruct((er_rows, 128), jnp.int32)),
        mesh=_mesh(),
        scratch_types=[
            pltpu.VMEM((CH, 128), jnp.int32),
            pltpu.VMEM((CH, 128), jnp.int32),
            pltpu.VMEM((CH, 128), jnp.int32),
            pltpu.VMEM((CH, 128), jnp.float32),
            pltpu.VMEM((CH, 128), jnp.int32),
            pltpu.VMEM((SEGP,), jnp.float32),
        ],
    )
    def sc_wgt(src_hbm, dst_hbm, et_hbm, inv_hbm, w_hbm, g_hbm,
               src_v, dst_v, et_v, w_v, g_v, inv_v):
        c = lax.axis_index("c")
        s = lax.axis_index("s")
        wid = c * NS + s

        pltpu.sync_copy(inv_hbm, inv_v)
        row0 = wid * rows_per_w

        @pl.loop(0, n_chunks)
        def _chunk(k):
            base = row0 + k * CH
            pltpu.sync_copy(src_hbm.at[pl.ds(base, CH)], src_v)
            pltpu.sync_copy(dst_hbm.at[pl.ds(base, CH)], dst_v)
            pltpu.sync_copy(et_hbm.at[pl.ds(base, CH)], et_v)
            for jr in range(CH):
                for jj in range(128 // L):
                    sl = pl.ds(jj * L, L)
                    e16 = et_v[jr, sl]
                    seg16 = dst_v[jr, sl] * R + e16
                    w_v[jr, sl] = plsc.load_gather(inv_v, [seg16])
                    g_v[jr, sl] = e16 * N + src_v[jr, sl]
            pltpu.sync_copy(w_v, w_hbm.at[pl.ds(base, CH)])
            pltpu.sync_copy(g_v, g_hbm.at[pl.ds(base, CH)])

    return sc_wgt


def _make_sc_msg(er_rows, rows_per_w):
    n_chunks = rows_per_w // CH
    acc_t = N_ACC // NS
    zr = acc_t // 2

    @functools.partial(
        pl.kernel,
        out_type=jax.ShapeDtypeStruct((NC * N_ACC, 128), jnp.float32),
        mesh=_mesh(),
        scratch_types=[
            pltpu.VMEM((CH, 128), jnp.int32),
            pltpu.VMEM((CH, 128), jnp.int32),
            pltpu.VMEM((CH, 128), jnp.float32),
            pltpu.VMEM((CH * 128, 128), jnp.float32),
            pltpu.VMEM((313, 128), jnp.float32),
            pltpu.VMEM_SHARED((N_ACC, 128), jnp.float32),
            pltpu.SemaphoreType.DMA,
        ],
    )
    def sc_msg(trans_hbm, g_hbm, dst_hbm, w_hbm, out_hbm,
               g_v, dst_v, w_v, rows_v, zer_v, acc_sh, sem):
        c = lax.axis_index("c")
        s = lax.axis_index("s")
        wid = c * NS + s

        _fill_zeros(zer_v, zr)
        pltpu.sync_copy(zer_v, acc_sh.at[pl.ds(s * acc_t, zr)])
        pltpu.sync_copy(zer_v, acc_sh.at[pl.ds(s * acc_t + zr, zr)])
        plsc.subcore_barrier()

        row0 = wid * rows_per_w

        @pl.loop(0, n_chunks)
        def _chunk(k):
            base = row0 + k * CH
            pltpu.sync_copy(g_hbm.at[pl.ds(base, CH)], g_v)
            pltpu.sync_copy(dst_hbm.at[pl.ds(base, CH)], dst_v)
            pltpu.sync_copy(w_hbm.at[pl.ds(base, CH)], w_v)
            cps = [pltpu.async_copy(trans_hbm.at[g_v.at[jr]],
                                    rows_v.at[pl.ds(jr * 128, 128)], sem)
                   for jr in range(CH)]
            for cp in cps:
                cp.wait()

            @pl.loop(0, CH * 128)
            def _scale(r):
                wj = w_v[r // 128, r % 128]
                for q in range(128 // L):
                    sl = pl.ds(q * L, L)
                    rows_v[r, sl] = rows_v[r, sl] * wj

            for jr in range(CH):
                pltpu.sync_copy(rows_v.at[pl.ds(jr * 128, 128)],
                                acc_sh.at[dst_v.at[jr]], add=True)

        plsc.subcore_barrier()
        pltpu.sync_copy(acc_sh.at[pl.ds(s * acc_t, acc_t)],
                        out_hbm.at[pl.ds(c * N_ACC + s * acc_t, acc_t)])

    return sc_msg


def _tc_inv(cnt):
    rows = SEGP // 128
    cnt3 = cnt.reshape(NC, rows, 128)

    def body(c_ref, o_ref):
        tot = c_ref[0] + c_ref[1]
        o_ref[...] = 1.0 / jnp.maximum(tot, 1.0)

    out = pl.pallas_call(
        body,
        out_shape=jax.ShapeDtypeStruct((rows, 128), jnp.float32),
    )(cnt3)
    return out.reshape(SEGP)


def _tc_trans(h, W, root, b):
    blk = 1000
    grid = (N // blk,)

    def body(h_ref, w_ref, r_ref, b_ref, t_ref, hr_ref):
        hh = h_ref[...]
        for r in range(R):
            t_ref[r] = jnp.dot(hh, w_ref[r], preferred_element_type=jnp.float32)
        hr_ref[...] = jnp.dot(hh, r_ref[...],
                              preferred_element_type=jnp.float32) + b_ref[...]

    return pl.pallas_call(
        body,
        grid=grid,
        in_specs=[
            pl.BlockSpec((blk, H), lambda i: (i, 0)),
            pl.BlockSpec((R, H, H), lambda i: (0, 0, 0)),
            pl.BlockSpec((H, H), lambda i: (0, 0)),
            pl.BlockSpec((1, H), lambda i: (0, 0)),
        ],
        out_specs=[
            pl.BlockSpec((R, blk, H), lambda i: (0, i, 0)),
            pl.BlockSpec((blk, H), lambda i: (i, 0)),
        ],
        out_shape=[
            jax.ShapeDtypeStruct((R, N, H), jnp.float32),
            jax.ShapeDtypeStruct((N, H), jnp.float32),
        ],
    )(h, W, root, b)


def _tc_combine(m0, m1, hroot):
    blk = 1000
    grid = (N // blk,)

    def body(a_ref, b_ref, c_ref, o_ref):
        o_ref[...] = jnp.maximum(a_ref[...] + b_ref[...] + c_ref[...], 0.0)

    return pl.pallas_call(
        body,
        grid=grid,
        in_specs=[pl.BlockSpec((blk, H), lambda i: (i, 0))] * 3,
        out_specs=pl.BlockSpec((blk, H), lambda i: (i, 0)),
        out_shape=jax.ShapeDtypeStruct((N, H), jnp.float32),
    )(m0, m1, hroot)


def _tc_set2set(h, batch2d, W_ih, W_hh, b_ih, b_hh):
    def body(h_ref, bt_ref, wih_ref, whh_ref, bih_ref, bhh_ref, out_ref):
        hh = h_ref[...]
        bt = bt_ref[...]
        iota = lax.broadcasted_iota(jnp.int32, (1, NB), 1)
        onehot = (bt == iota).astype(jnp.float32)

        wih = wih_ref[...]
        whh = whh_ref[...]
        bih = bih_ref[...]
        bhh = bhh_ref[...]

        q_star = jnp.zeros((NB, 2 * H), jnp.float32)
        hx = jnp.zeros((NB, H), jnp.float32)
        cx = jnp.zeros((NB, H), jnp.float32)

        for _ in range(STEPS):
            gates = (lax.dot_general(q_star, wih, (((1,), (1,)), ((), ())),
                                     preferred_element_type=jnp.float32)
                     + bih
                     + lax.dot_general(hx, whh, (((1,), (1,)), ((), ())),
                                       preferred_element_type=jnp.float32)
                     + bhh)
            gi = jax.nn.sigmoid(gates[:, 0 * H:1 * H])
            gf = jax.nn.sigmoid(gates[:, 1 * H:2 * H])
            gg = jnp.tanh(gates[:, 2 * H:3 * H])
            go = jax.nn.sigmoid(gates[:, 3 * H:4 * H])
            cx = gf * cx + gi * gg
            hx = go * jnp.tanh(cx)
            q = hx

            qb = jnp.dot(onehot, q, preferred_element_type=jnp.float32)
            e = jnp.sum(hh * qb, axis=1, keepdims=True)
            em = jnp.where(onehot > 0.0, e, -1e30)
            m = jnp.max(em, axis=0, keepdims=True)
            m = jnp.where(m > -9e29, m, 0.0)
            mb = jnp.sum(onehot * m, axis=1, keepdims=True)
            ex = jnp.exp(e - mb)
            den = jnp.sum(onehot * ex, axis=0, keepdims=True)
            denb = jnp.sum(onehot * den, axis=1, keepdims=True)
            a = ex / (denb + 1e-16)
            ma = onehot * a
            r = lax.dot_general(ma, hh, (((0,), (0,)), ((), ())),
                                preferred_element_type=jnp.float32)
            q_star = jnp.concatenate([q, r], axis=-1)

        out_ref[...] = q_star

    return pl.pallas_call(
        body,
        out_shape=jax.ShapeDtypeStruct((NB, 2 * H), jnp.float32),
    )(h, batch2d, W_ih, W_hh, b_ih, b_hh)


def kernel(x, edge_index, edge_type, batch, W0, root0, b0, W1, root1, b1,
           W2, root2, b2, W_ih, W_hh, b_ih, b_hh):
    E = edge_index.shape[1]

    rows_per_w = _cdiv(_cdiv(E, 128), NW * CH) * CH
    er_rows = NW * rows_per_w
    ep = er_rows * 128
    pad = ep - E

    src = edge_index[0]
    dst = edge_index[1]
    src_p = jnp.concatenate([src, jnp.zeros((pad,), jnp.int32)]).reshape(er_rows, 128)
    dst_p = jnp.concatenate([dst, jnp.full((pad,), PAD_DST, jnp.int32)]).reshape(er_rows, 128)
    et_p = jnp.concatenate([edge_type, jnp.zeros((pad,), jnp.int32)]).reshape(er_rows, 128)

    cnt = _make_sc_cnt(er_rows, rows_per_w)(dst_p, et_p)
    inv = _tc_inv(cnt.reshape(NC, SEGP))
    wgt, gidx = _make_sc_wgt(er_rows, rows_per_w)(src_p, dst_p, et_p, inv)

    sc_msg = _make_sc_msg(er_rows, rows_per_w)

    h = x
    for (W, root, b) in ((W0, root0, b0), (W1, root1, b1), (W2, root2, b2)):
        trans, hroot = _tc_trans(h, W, root, b.reshape(1, H))
        msg = sc_msg(trans.reshape(R * N, H), gidx, dst_p, wgt)
        msg = msg.reshape(NC, N_ACC, H)
        h = _tc_combine(msg[0, :N], msg[1, :N], hroot)

    emb = _tc_set2set(h, batch.reshape(N, 1), W_ih, W_hh,
                      b_ih.reshape(1, 4 * H), b_hh.reshape(1, 4 * H))
    return (h, emb)


# speedup vs baseline: 3.1743x; 3.1743x over previous
import functools

import jax
import jax.numpy as jnp
from jax import lax
from jax.experimental import pallas as pl
from jax.experimental.pallas import tpu as pltpu
from jax.experimental.pallas import tpu_sc as plsc

NC = 2
NS = 16
NW = NC * NS
L = 16

N = 10000
R = 8
H = 128
NB = 64
STEPS = 3

CH = 8
SEGP = 81920
SEG_T = SEGP // NS
N_ACC = 10112
PAD_DST = N + 8


def _cdiv(a, b):
    return (a + b - 1) // b


def _mesh():
    return plsc.VectorSubcoreMesh(
        core_axis_name="c", subcore_axis_name="s", num_cores=NC, num_subcores=NS)


def _fill_zeros(ref, nrows):
    zero = jnp.zeros((L,), jnp.float32)

    @pl.loop(0, nrows)
    def _row(r):
        for j in range(128 // L):
            ref[r, pl.ds(j * L, L)] = zero


def _make_sc_cnt(er_rows, rows_per_w):
    n_chunks = rows_per_w // CH

    @functools.partial(
        pl.kernel,
        out_type=jax.ShapeDtypeStruct((NC * SEGP,), jnp.float32),
        mesh=_mesh(),
        scratch_types=[
            pltpu.VMEM((CH, 128), jnp.int32),
            pltpu.VMEM((CH, 128), jnp.int32),
            pltpu.VMEM((CH, 128), jnp.int32),
            pltpu.VMEM((128,), jnp.float32),
            pltpu.VMEM((SEG_T,), jnp.float32),
            pltpu.VMEM_SHARED((SEGP,), jnp.float32),
        ],
    )
    def sc_cnt(dst_hbm, et_hbm, out_hbm, dst_v, et_v, seg_v, ones_v, zer_v, cnt_sh):
        c = lax.axis_index("c")
        s = lax.axis_index("s")
        wid = c * NS + s

        zero = jnp.zeros((L,), jnp.float32)
        one = jnp.ones((L,), jnp.float32)

        @pl.loop(0, SEG_T // L)
        def _z(i):
            zer_v[pl.ds(i * L, L)] = zero

        for j in range(128 // L):
            ones_v[pl.ds(j * L, L)] = one

        pltpu.sync_copy(zer_v, cnt_sh.at[pl.ds(s * SEG_T, SEG_T)])
        plsc.subcore_barrier()

        row0 = wid * rows_per_w

        @pl.loop(0, n_chunks)
        def _chunk(k):
            base = row0 + k * CH
            pltpu.sync_copy(dst_hbm.at[pl.ds(base, CH)], dst_v)
            pltpu.sync_copy(et_hbm.at[pl.ds(base, CH)], et_v)
            for jr in range(CH):
                for jj in range(128 // L):
                    sl = pl.ds(jj * L, L)
                    seg_v[jr, sl] = dst_v[jr, sl] * R + et_v[jr, sl]
            for jr in range(CH):
                pltpu.sync_copy(ones_v, cnt_sh.at[seg_v.at[jr]], add=True)

        plsc.subcore_barrier()
        pltpu.sync_copy(cnt_sh.at[pl.ds(s * SEG_T, SEG_T)], zer_v)
        pltpu.sync_copy(zer_v, out_hbm.at[pl.ds(c * SEGP + s * SEG_T, SEG_T)])

    return sc_cnt


def _make_sc_wgt(er_rows, rows_per_w):
    n_chunks = rows_per_w // CH

    @functools.partial(
        pl.kernel,
        out_type=(jax.ShapeDtypeStruct((er_rows, 128), jnp.float32),
                  jax.ShapeDtypeStruct((er_rows, 128), jnp.int32)),
        mesh=_mesh(),
        compiler_params=pltpu.CompilerParams(needs_layout_passes=False),
        scratch_types=[
            pltpu.VMEM((CH, 128), jnp.int32),
            pltpu.VMEM((CH, 128), jnp.int32),
            pltpu.VMEM((CH, 128), jnp.int32),
            pltpu.VMEM((CH, 128), jnp.float32),
            pltpu.VMEM((CH, 128), jnp.int32),
            pltpu.VMEM((SEGP,), jnp.float32),
        ],
    )
    def sc_wgt(src_hbm, dst_hbm, et_hbm, inv_hbm, w_hbm, g_hbm,
               src_v, dst_v, et_v, w_v, g_v, inv_v):
        c = lax.axis_index("c")
        s = lax.axis_index("s")
        wid = c * NS + s

        pltpu.sync_copy(inv_hbm, inv_v)
        row0 = wid * rows_per_w

        @pl.loop(0, n_chunks)
        def _chunk(k):
            base = row0 + k * CH
            pltpu.sync_copy(src_hbm.at[pl.ds(base, CH)], src_v)
            pltpu.sync_copy(dst_hbm.at[pl.ds(base, CH)], dst_v)
            pltpu.sync_copy(et_hbm.at[pl.ds(base, CH)], et_v)
            for jr in range(CH):
                for jj in range(128 // L):
                    sl = pl.ds(jj * L, L)
                    e16 = et_v.at[jr][sl]
                    seg16 = dst_v.at[jr][sl] * R + e16
                    w_v.at[jr][sl] = plsc.load_gather(inv_v, [seg16])
                    g_v.at[jr][sl] = e16 * N + src_v.at[jr][sl]
            pltpu.sync_copy(w_v, w_hbm.at[pl.ds(base, CH)])
            pltpu.sync_copy(g_v, g_hbm.at[pl.ds(base, CH)])

    return sc_wgt


def _make_sc_msg(er_rows, rows_per_w):
    n_chunks = rows_per_w // CH
    acc_t = N_ACC // NS
    ZR = 64

    @functools.partial(
        pl.kernel,
        out_type=jax.ShapeDtypeStruct((NC * N_ACC, 128), jnp.float32),
        mesh=_mesh(),
        scratch_types=[
            pltpu.VMEM((CH, 128), jnp.int32),
            pltpu.VMEM((CH, 128), jnp.int32),
            pltpu.VMEM((CH, 128), jnp.float32),
            pltpu.VMEM((128, 128), jnp.float32),
            pltpu.VMEM((ZR, 128), jnp.float32),
            pltpu.VMEM_SHARED((N_ACC, 128), jnp.float32),
            pltpu.SemaphoreType.DMA,
        ],
    )
    def sc_msg(trans_hbm, g_hbm, dst_hbm, w_hbm, out_hbm,
               g_v, dst_v, w_v, rows_v, zer_v, acc_sh, sem):
        c = lax.axis_index("c")
        s = lax.axis_index("s")
        wid = c * NS + s

        _fill_zeros(zer_v, ZR)
        for i in range(9):
            pltpu.sync_copy(zer_v, acc_sh.at[pl.ds(s * acc_t + i * ZR, ZR)])
        pltpu.sync_copy(zer_v.at[pl.ds(0, acc_t - 9 * ZR)],
                        acc_sh.at[pl.ds(s * acc_t + 9 * ZR, acc_t - 9 * ZR)])
        plsc.subcore_barrier()

        row0 = wid * rows_per_w

        @pl.loop(0, n_chunks)
        def _chunk(k):
            base = row0 + k * CH
            pltpu.sync_copy(g_hbm.at[pl.ds(base, CH)], g_v)
            pltpu.sync_copy(dst_hbm.at[pl.ds(base, CH)], dst_v)
            pltpu.sync_copy(w_hbm.at[pl.ds(base, CH)], w_v)
            for jr in range(CH):
                pltpu.async_copy(trans_hbm.at[g_v.at[jr]], rows_v, sem).wait()

                @pl.loop(0, 128 // L)
                def _scale(g):
                    jc = g * L
                    w16 = w_v[jr, pl.ds(jc, L)]
                    r0 = g * L
                    for t in range(L):
                        wj = w16[t]
                        for q in range(128 // L):
                            sl = pl.ds(q * L, L)
                            rows_v[r0 + t, sl] = rows_v[r0 + t, sl] * wj

                pltpu.sync_copy(rows_v, acc_sh.at[dst_v.at[jr]], add=True)

        plsc.subcore_barrier()
        for i in range(9):
            pltpu.sync_copy(acc_sh.at[pl.ds(s * acc_t + i * ZR, ZR)], zer_v)
            pltpu.sync_copy(zer_v,
                            out_hbm.at[pl.ds(c * N_ACC + s * acc_t + i * ZR, ZR)])
        rem = acc_t - 9 * ZR
        pltpu.sync_copy(acc_sh.at[pl.ds(s * acc_t + 9 * ZR, rem)],
                        zer_v.at[pl.ds(0, rem)])
        pltpu.sync_copy(zer_v.at[pl.ds(0, rem)],
                        out_hbm.at[pl.ds(c * N_ACC + s * acc_t + 9 * ZR, rem)])

    return sc_msg


def _tc_inv(cnt):
    rows = SEGP // 128
    cnt3 = cnt.reshape(NC, rows, 128)

    def body(c_ref, o_ref):
        tot = c_ref[0] + c_ref[1]
        o_ref[...] = 1.0 / jnp.maximum(tot, 1.0)

    out = pl.pallas_call(
        body,
        out_shape=jax.ShapeDtypeStruct((rows, 128), jnp.float32),
    )(cnt3)
    return out.reshape(SEGP)


def _tc_trans(h, W, root, b):
    blk = 1000
    grid = (N // blk,)

    def body(h_ref, w_ref, r_ref, b_ref, t_ref, hr_ref):
        hh = h_ref[...]
        for r in range(R):
            t_ref[r] = jnp.dot(hh, w_ref[r], preferred_element_type=jnp.float32)
        hr_ref[...] = jnp.dot(hh, r_ref[...],
                              preferred_element_type=jnp.float32) + b_ref[...]

    return pl.pallas_call(
        body,
        grid=grid,
        in_specs=[
            pl.BlockSpec((blk, H), lambda i: (i, 0)),
            pl.BlockSpec((R, H, H), lambda i: (0, 0, 0)),
            pl.BlockSpec((H, H), lambda i: (0, 0)),
            pl.BlockSpec((1, H), lambda i: (0, 0)),
        ],
        out_specs=[
            pl.BlockSpec((R, blk, H), lambda i: (0, i, 0)),
            pl.BlockSpec((blk, H), lambda i: (i, 0)),
        ],
        out_shape=[
            jax.ShapeDtypeStruct((R, N, H), jnp.float32),
            jax.ShapeDtypeStruct((N, H), jnp.float32),
        ],
    )(h, W, root, b)


def _tc_combine(m0, m1, hroot):
    blk = 1000
    grid = (N // blk,)

    def body(a_ref, b_ref, c_ref, o_ref):
        o_ref[...] = jnp.maximum(a_ref[...] + b_ref[...] + c_ref[...], 0.0)

    return pl.pallas_call(
        body,
        grid=grid,
        in_specs=[pl.BlockSpec((blk, H), lambda i: (i, 0))] * 3,
        out_specs=pl.BlockSpec((blk, H), lambda i: (i, 0)),
        out_shape=jax.ShapeDtypeStruct((N, H), jnp.float32),
    )(m0, m1, hroot)


def _tc_set2set(h, batch2d, W_ih, W_hh, b_ih, b_hh):
    def body(h_ref, bt_ref, wih_ref, whh_ref, bih_ref, bhh_ref, out_ref):
        hh = h_ref[...]
        bt = bt_ref[...]
        iota = lax.broadcasted_iota(jnp.int32, (1, NB), 1)
        onehot = (bt == iota).astype(jnp.float32)

        wih = wih_ref[...]
        whh = whh_ref[...]
        bih = bih_ref[...]
        bhh = bhh_ref[...]

        q_star = jnp.zeros((NB, 2 * H), jnp.float32)
        hx = jnp.zeros((NB, H), jnp.float32)
        cx = jnp.zeros((NB, H), jnp.float32)

        for _ in range(STEPS):
            gates = (lax.dot_general(q_star, wih, (((1,), (1,)), ((), ())),
                                     preferred_element_type=jnp.float32)
                     + bih
                     + lax.dot_general(hx, whh, (((1,), (1,)), ((), ())),
                                       preferred_element_type=jnp.float32)
                     + bhh)
            gi = jax.nn.sigmoid(gates[:, 0 * H:1 * H])
            gf = jax.nn.sigmoid(gates[:, 1 * H:2 * H])
            gg = jnp.tanh(gates[:, 2 * H:3 * H])
            go = jax.nn.sigmoid(gates[:, 3 * H:4 * H])
            cx = gf * cx + gi * gg
            hx = go * jnp.tanh(cx)
            q = hx

            qb = jnp.dot(onehot, q, preferred_element_type=jnp.float32)
            e = jnp.sum(hh * qb, axis=1, keepdims=True)
            em = jnp.where(onehot > 0.0, e, -1e30)
            m = jnp.max(em, axis=0, keepdims=True)
            m = jnp.where(m > -9e29, m, 0.0)
            mb = jnp.sum(onehot * m, axis=1, keepdims=True)
            ex = jnp.exp(e - mb)
            den = jnp.sum(onehot * ex, axis=0, keepdims=True)
            denb = jnp.sum(onehot * den, axis=1, keepdims=True)
            a = ex / (denb + 1e-16)
            ma = onehot * a
            r = lax.dot_general(ma, hh, (((0,), (0,)), ((), ())),
                                preferred_element_type=jnp.float32)
            q_star = jnp.concatenate([q, r], axis=-1)

        out_ref[...] = q_star

    return pl.pallas_call(
        body,
        out_shape=jax.ShapeDtypeStruct((NB, 2 * H), jnp.float32),
    )(h, batch2d, W_ih, W_hh, b_ih, b_hh)


def kernel(x, edge_index, edge_type, batch, W0, root0, b0, W1, root1, b1,
           W2, root2, b2, W_ih, W_hh, b_ih, b_hh):
    E = edge_index.shape[1]

    rows_per_w = _cdiv(_cdiv(E, 128), NW * CH) * CH
    er_rows = NW * rows_per_w
    ep = er_rows * 128
    pad = ep - E

    src = edge_index[0]
    dst = edge_index[1]
    src_p = jnp.concatenate([src, jnp.zeros((pad,), jnp.int32)]).reshape(er_rows, 128)
    dst_p = jnp.concatenate([dst, jnp.full((pad,), PAD_DST, jnp.int32)]).reshape(er_rows, 128)
    et_p = jnp.concatenate([edge_type, jnp.zeros((pad,), jnp.int32)]).reshape(er_rows, 128)

    cnt = _make_sc_cnt(er_rows, rows_per_w)(dst_p, et_p)
    inv = _tc_inv(cnt.reshape(NC, SEGP))
    wgt, gidx = _make_sc_wgt(er_rows, rows_per_w)(src_p, dst_p, et_p, inv)

    sc_msg = _make_sc_msg(er_rows, rows_per_w)

    h = x
    for (W, root, b) in ((W0, root0, b0), (W1, root1, b1), (W2, root2, b2)):
        trans, hroot = _tc_trans(h, W, root, b.reshape(1, H))
        msg = sc_msg(trans.reshape(R * N, H), gidx, dst_p, wgt)
        msg = msg.reshape(NC, N_ACC, H)
        h = _tc_combine(msg[0, :N], msg[1, :N], hroot)

    emb = _tc_set2set(h, batch.reshape(N, 1), W_ih, W_hh,
                      b_ih.reshape(1, 4 * H), b_hh.reshape(1, 4 * H))
    return (h, emb)


# speedup vs baseline: 3.6482x; 1.1493x over previous
import functools

import jax
import jax.numpy as jnp
from jax import lax
from jax.experimental import pallas as pl
from jax.experimental.pallas import tpu as pltpu
from jax.experimental.pallas import tpu_sc as plsc

NC = 2
NS = 16
NW = NC * NS
L = 16

N = 10000
R = 8
H = 128
NB = 64
STEPS = 3

CH = 8
SEGP = 81920
SEG_T = SEGP // NS
N_ACC = 10112
PAD_DST = N + 8


def _cdiv(a, b):
    return (a + b - 1) // b


def _mesh():
    return plsc.VectorSubcoreMesh(
        core_axis_name="c", subcore_axis_name="s", num_cores=NC, num_subcores=NS)


def _fill_zeros(ref, nrows):
    zero = jnp.zeros((L,), jnp.float32)

    @pl.loop(0, nrows)
    def _row(r):
        for j in range(128 // L):
            ref[r, pl.ds(j * L, L)] = zero


def _make_sc_cnt(er_rows, rows_per_w):
    n_chunks = rows_per_w // CH

    @functools.partial(
        pl.kernel,
        out_type=jax.ShapeDtypeStruct((NC * SEGP,), jnp.float32),
        mesh=_mesh(),
        scratch_types=[
            pltpu.VMEM((CH, 128), jnp.int32),
            pltpu.VMEM((CH, 128), jnp.int32),
            pltpu.VMEM((CH, 128), jnp.int32),
            pltpu.VMEM((128,), jnp.float32),
            pltpu.VMEM((SEG_T,), jnp.float32),
            pltpu.VMEM_SHARED((SEGP,), jnp.float32),
        ],
    )
    def sc_cnt(dst_hbm, et_hbm, out_hbm, dst_v, et_v, seg_v, ones_v, zer_v, cnt_sh):
        c = lax.axis_index("c")
        s = lax.axis_index("s")
        wid = c * NS + s

        zero = jnp.zeros((L,), jnp.float32)
        one = jnp.ones((L,), jnp.float32)

        @pl.loop(0, SEG_T // L)
        def _z(i):
            zer_v[pl.ds(i * L, L)] = zero

        for j in range(128 // L):
            ones_v[pl.ds(j * L, L)] = one

        pltpu.sync_copy(zer_v, cnt_sh.at[pl.ds(s * SEG_T, SEG_T)])
        plsc.subcore_barrier()

        row0 = wid * rows_per_w

        @pl.loop(0, n_chunks)
        def _chunk(k):
            base = row0 + k * CH
            pltpu.sync_copy(dst_hbm.at[pl.ds(base, CH)], dst_v)
            pltpu.sync_copy(et_hbm.at[pl.ds(base, CH)], et_v)
            for jr in range(CH):
                for jj in range(128 // L):
                    sl = pl.ds(jj * L, L)
                    seg_v[jr, sl] = dst_v[jr, sl] * R + et_v[jr, sl]
            for jr in range(CH):
                pltpu.sync_copy(ones_v, cnt_sh.at[seg_v.at[jr]], add=True)

        plsc.subcore_barrier()
        pltpu.sync_copy(cnt_sh.at[pl.ds(s * SEG_T, SEG_T)], zer_v)
        pltpu.sync_copy(zer_v, out_hbm.at[pl.ds(c * SEGP + s * SEG_T, SEG_T)])

    return sc_cnt


def _make_sc_wgt(er_rows, rows_per_w):
    n_chunks = rows_per_w // CH

    @functools.partial(
        pl.kernel,
        out_type=(jax.ShapeDtypeStruct((er_rows, 128), jnp.float32),
                  jax.ShapeDtypeStruct((er_rows, 128), jnp.int32)),
        mesh=_mesh(),
        compiler_params=pltpu.CompilerParams(needs_layout_passes=False),
        scratch_types=[
            pltpu.VMEM((CH, 128), jnp.int32),
            pltpu.VMEM((CH, 128), jnp.int32),
            pltpu.VMEM((CH, 128), jnp.int32),
            pltpu.VMEM((CH, 128), jnp.float32),
            pltpu.VMEM((CH, 128), jnp.int32),
            pltpu.VMEM((SEGP,), jnp.float32),
        ],
    )
    def sc_wgt(src_hbm, dst_hbm, et_hbm, inv_hbm, w_hbm, g_hbm,
               src_v, dst_v, et_v, w_v, g_v, inv_v):
        c = lax.axis_index("c")
        s = lax.axis_index("s")
        wid = c * NS + s

        pltpu.sync_copy(inv_hbm, inv_v)
        row0 = wid * rows_per_w

        @pl.loop(0, n_chunks)
        def _chunk(k):
            base = row0 + k * CH
            pltpu.sync_copy(src_hbm.at[pl.ds(base, CH)], src_v)
            pltpu.sync_copy(dst_hbm.at[pl.ds(base, CH)], dst_v)
            pltpu.sync_copy(et_hbm.at[pl.ds(base, CH)], et_v)
            for jr in range(CH):
                for jj in range(128 // L):
                    sl = pl.ds(jj * L, L)
                    e16 = et_v.at[jr][sl]
                    seg16 = dst_v.at[jr][sl] * R + e16
                    w_v.at[jr][sl] = plsc.load_gather(inv_v, [seg16])
                    g_v.at[jr][sl] = e16 * N + src_v.at[jr][sl]
            pltpu.sync_copy(w_v, w_hbm.at[pl.ds(base, CH)])
            pltpu.sync_copy(g_v, g_hbm.at[pl.ds(base, CH)])

    return sc_wgt


def _make_sc_msg(er_rows, rows_per_w):
    n_chunks = rows_per_w // CH
    acc_t = N_ACC // NS

    def _scale_rows(rows_v, w_v, jr):
        @pl.loop(0, 128 // L)
        def _scale(g):
            jc = g * L
            w16 = w_v[jr, pl.ds(jc, L)]
            r0 = g * L
            for t in range(L):
                wj = w16[t]
                for q in range(128 // L):
                    sl = pl.ds(q * L, L)
                    rows_v[r0 + t, sl] = rows_v[r0 + t, sl] * wj

    @functools.partial(
        pl.kernel,
        out_type=jax.ShapeDtypeStruct((NC * N_ACC, 128), jnp.float32),
        mesh=_mesh(),
        scratch_types=[
            pltpu.VMEM((CH, 128), jnp.int32),
            pltpu.VMEM((CH, 128), jnp.int32),
            pltpu.VMEM((CH, 128), jnp.float32),
            pltpu.VMEM((128, 128), jnp.float32),
            pltpu.VMEM((128, 128), jnp.float32),
            pltpu.VMEM_SHARED((N_ACC, 128), jnp.float32),
            pltpu.SemaphoreType.DMA,
            pltpu.SemaphoreType.DMA,
        ],
    )
    def sc_msg(trans_hbm, g_hbm, dst_hbm, w_hbm, out_hbm,
               g_v, dst_v, w_v, rows_a, rows_b, acc_sh, sem_a, sem_b):
        c = lax.axis_index("c")
        s = lax.axis_index("s")
        wid = c * NS + s
        bufs = (rows_a, rows_b)
        sems = (sem_a, sem_b)

        _fill_zeros(rows_a, 128)
        for i in range(4):
            pltpu.sync_copy(rows_a, acc_sh.at[pl.ds(s * acc_t + i * 128, 128)])
        rem0 = acc_t - 4 * 128
        pltpu.sync_copy(rows_a.at[pl.ds(0, rem0)],
                        acc_sh.at[pl.ds(s * acc_t + 4 * 128, rem0)])
        plsc.subcore_barrier()

        row0 = wid * rows_per_w

        @pl.loop(0, n_chunks)
        def _chunk(k):
            base = row0 + k * CH
            pltpu.sync_copy(g_hbm.at[pl.ds(base, CH)], g_v)
            pltpu.sync_copy(dst_hbm.at[pl.ds(base, CH)], dst_v)
            pltpu.sync_copy(w_hbm.at[pl.ds(base, CH)], w_v)
            cps = [None] * CH
            cps[0] = pltpu.async_copy(trans_hbm.at[g_v.at[0]], bufs[0], sems[0])
            for jr in range(CH):
                sl = jr % 2
                if jr + 1 < CH:
                    cps[jr + 1] = pltpu.async_copy(
                        trans_hbm.at[g_v.at[jr + 1]], bufs[1 - sl], sems[1 - sl])
                cps[jr].wait()
                _scale_rows(bufs[sl], w_v, jr)
                pltpu.sync_copy(bufs[sl], acc_sh.at[dst_v.at[jr]], add=True)

        plsc.subcore_barrier()
        for i in range(4):
            pltpu.sync_copy(acc_sh.at[pl.ds(s * acc_t + i * 128, 128)], rows_a)
            pltpu.sync_copy(rows_a,
                            out_hbm.at[pl.ds(c * N_ACC + s * acc_t + i * 128, 128)])
        rem = acc_t - 4 * 128
        pltpu.sync_copy(acc_sh.at[pl.ds(s * acc_t + 4 * 128, rem)],
                        rows_a.at[pl.ds(0, rem)])
        pltpu.sync_copy(rows_a.at[pl.ds(0, rem)],
                        out_hbm.at[pl.ds(c * N_ACC + s * acc_t + 4 * 128, rem)])

    return sc_msg


def _tc_inv(cnt):
    rows = SEGP // 128
    cnt3 = cnt.reshape(NC, rows, 128)

    def body(c_ref, o_ref):
        tot = c_ref[0] + c_ref[1]
        o_ref[...] = 1.0 / jnp.maximum(tot, 1.0)

    out = pl.pallas_call(
        body,
        out_shape=jax.ShapeDtypeStruct((rows, 128), jnp.float32),
    )(cnt3)
    return out.reshape(SEGP)


def _tc_trans(h, W, root, b):
    blk = 1000
    grid = (N // blk,)

    def body(h_ref, w_ref, r_ref, b_ref, t_ref, hr_ref):
        hh = h_ref[...]
        for r in range(R):
            t_ref[r] = jnp.dot(hh, w_ref[r], preferred_element_type=jnp.float32)
        hr_ref[...] = jnp.dot(hh, r_ref[...],
                              preferred_element_type=jnp.float32) + b_ref[...]

    return pl.pallas_call(
        body,
        grid=grid,
        in_specs=[
            pl.BlockSpec((blk, H), lambda i: (i, 0)),
            pl.BlockSpec((R, H, H), lambda i: (0, 0, 0)),
            pl.BlockSpec((H, H), lambda i: (0, 0)),
            pl.BlockSpec((1, H), lambda i: (0, 0)),
        ],
        out_specs=[
            pl.BlockSpec((R, blk, H), lambda i: (0, i, 0)),
            pl.BlockSpec((blk, H), lambda i: (i, 0)),
        ],
        out_shape=[
            jax.ShapeDtypeStruct((R, N, H), jnp.float32),
            jax.ShapeDtypeStruct((N, H), jnp.float32),
        ],
    )(h, W, root, b)


def _tc_combine(m0, m1, hroot):
    blk = 1000
    grid = (N // blk,)

    def body(a_ref, b_ref, c_ref, o_ref):
        o_ref[...] = jnp.maximum(a_ref[...] + b_ref[...] + c_ref[...], 0.0)

    return pl.pallas_call(
        body,
        grid=grid,
        in_specs=[pl.BlockSpec((blk, H), lambda i: (i, 0))] * 3,
        out_specs=pl.BlockSpec((blk, H), lambda i: (i, 0)),
        out_shape=jax.ShapeDtypeStruct((N, H), jnp.float32),
    )(m0, m1, hroot)


def _tc_set2set(h, batch2d, W_ih, W_hh, b_ih, b_hh):
    def body(h_ref, bt_ref, wih_ref, whh_ref, bih_ref, bhh_ref, out_ref):
        hh = h_ref[...]
        bt = bt_ref[...]
        iota = lax.broadcasted_iota(jnp.int32, (1, NB), 1)
        onehot = (bt == iota).astype(jnp.float32)

        wih = wih_ref[...]
        whh = whh_ref[...]
        bih = bih_ref[...]
        bhh = bhh_ref[...]

        q_star = jnp.zeros((NB, 2 * H), jnp.float32)
        hx = jnp.zeros((NB, H), jnp.float32)
        cx = jnp.zeros((NB, H), jnp.float32)

        for _ in range(STEPS):
            gates = (lax.dot_general(q_star, wih, (((1,), (1,)), ((), ())),
                                     preferred_element_type=jnp.float32)
                     + bih
                     + lax.dot_general(hx, whh, (((1,), (1,)), ((), ())),
                                       preferred_element_type=jnp.float32)
                     + bhh)
            gi = jax.nn.sigmoid(gates[:, 0 * H:1 * H])
            gf = jax.nn.sigmoid(gates[:, 1 * H:2 * H])
            gg = jnp.tanh(gates[:, 2 * H:3 * H])
            go = jax.nn.sigmoid(gates[:, 3 * H:4 * H])
            cx = gf * cx + gi * gg
            hx = go * jnp.tanh(cx)
            q = hx

            qb = jnp.dot(onehot, q, preferred_element_type=jnp.float32)
            e = jnp.sum(hh * qb, axis=1, keepdims=True)
            em = jnp.where(onehot > 0.0, e, -1e30)
            m = jnp.max(em, axis=0, keepdims=True)
            m = jnp.where(m > -9e29, m, 0.0)
            mb = jnp.sum(onehot * m, axis=1, keepdims=True)
            ex = jnp.exp(e - mb)
            den = jnp.sum(onehot * ex, axis=0, keepdims=True)
            denb = jnp.sum(onehot * den, axis=1, keepdims=True)
            a = ex / (denb + 1e-16)
            ma = onehot * a
            r = lax.dot_general(ma, hh, (((0,), (0,)), ((), ())),
                                preferred_element_type=jnp.float32)
            q_star = jnp.concatenate([q, r], axis=-1)

        out_ref[...] = q_star

    return pl.pallas_call(
        body,
        out_shape=jax.ShapeDtypeStruct((NB, 2 * H), jnp.float32),
    )(h, batch2d, W_ih, W_hh, b_ih, b_hh)


def kernel(x, edge_index, edge_type, batch, W0, root0, b0, W1, root1, b1,
           W2, root2, b2, W_ih, W_hh, b_ih, b_hh):
    E = edge_index.shape[1]

    rows_per_w = _cdiv(_cdiv(E, 128), NW * CH) * CH
    er_rows = NW * rows_per_w
    ep = er_rows * 128
    pad = ep - E

    src = edge_index[0]
    dst = edge_index[1]
    src_p = jnp.concatenate([src, jnp.zeros((pad,), jnp.int32)]).reshape(er_rows, 128)
    dst_p = jnp.concatenate([dst, jnp.full((pad,), PAD_DST, jnp.int32)]).reshape(er_rows, 128)
    et_p = jnp.concatenate([edge_type, jnp.zeros((pad,), jnp.int32)]).reshape(er_rows, 128)

    cnt = _make_sc_cnt(er_rows, rows_per_w)(dst_p, et_p)
    inv = _tc_inv(cnt.reshape(NC, SEGP))
    wgt, gidx = _make_sc_wgt(er_rows, rows_per_w)(src_p, dst_p, et_p, inv)

    sc_msg = _make_sc_msg(er_rows, rows_per_w)

    h = x
    for (W, root, b) in ((W0, root0, b0), (W1, root1, b1), (W2, root2, b2)):
        trans, hroot = _tc_trans(h, W, root, b.reshape(1, H))
        msg = sc_msg(trans.reshape(R * N, H), gidx, dst_p, wgt)
        msg = msg.reshape(NC, N_ACC, H)
        h = _tc_combine(msg[0, :N], msg[1, :N], hroot)

    emb = _tc_set2set(h, batch.reshape(N, 1), W_ih, W_hh,
                      b_ih.reshape(1, 4 * H), b_hh.reshape(1, 4 * H))
    return (h, emb)


# speedup vs baseline: 3.6871x; 1.0107x over previous
import functools

import jax
import jax.numpy as jnp
from jax import lax
from jax.experimental import pallas as pl
from jax.experimental.pallas import tpu as pltpu
from jax.experimental.pallas import tpu_sc as plsc

NC = 2
NS = 16
NW = NC * NS
L = 16

N = 10000
R = 8
H = 128
NB = 64
STEPS = 3

CH = 8
SEGP = 81920
SEG_T = SEGP // NS
N_ACC = 10112
PAD_DST = N + 8


def _cdiv(a, b):
    return (a + b - 1) // b


def _mesh():
    return plsc.VectorSubcoreMesh(
        core_axis_name="c", subcore_axis_name="s", num_cores=NC, num_subcores=NS)


def _fill_zeros(ref, nrows):
    zero = jnp.zeros((L,), jnp.float32)

    @pl.loop(0, nrows)
    def _row(r):
        for j in range(128 // L):
            ref[r, pl.ds(j * L, L)] = zero


def _make_sc_cnt(er_rows, rows_per_w):
    n_chunks = rows_per_w // CH

    @functools.partial(
        pl.kernel,
        out_type=jax.ShapeDtypeStruct((NC * SEGP,), jnp.float32),
        mesh=_mesh(),
        scratch_types=[
            pltpu.VMEM((CH, 128), jnp.int32),
            pltpu.VMEM((CH, 128), jnp.int32),
            pltpu.VMEM((CH, 128), jnp.int32),
            pltpu.VMEM((128,), jnp.float32),
            pltpu.VMEM((SEG_T,), jnp.float32),
            pltpu.VMEM_SHARED((SEGP,), jnp.float32),
        ],
    )
    def sc_cnt(dst_hbm, et_hbm, out_hbm, dst_v, et_v, seg_v, ones_v, zer_v, cnt_sh):
        c = lax.axis_index("c")
        s = lax.axis_index("s")
        wid = c * NS + s

        zero = jnp.zeros((L,), jnp.float32)
        one = jnp.ones((L,), jnp.float32)

        @pl.loop(0, SEG_T // L)
        def _z(i):
            zer_v[pl.ds(i * L, L)] = zero

        for j in range(128 // L):
            ones_v[pl.ds(j * L, L)] = one

        pltpu.sync_copy(zer_v, cnt_sh.at[pl.ds(s * SEG_T, SEG_T)])
        plsc.subcore_barrier()

        row0 = wid * rows_per_w

        @pl.loop(0, n_chunks)
        def _chunk(k):
            base = row0 + k * CH
            pltpu.sync_copy(dst_hbm.at[pl.ds(base, CH)], dst_v)
            pltpu.sync_copy(et_hbm.at[pl.ds(base, CH)], et_v)
            for jr in range(CH):
                for jj in range(128 // L):
                    sl = pl.ds(jj * L, L)
                    seg_v[jr, sl] = dst_v[jr, sl] * R + et_v[jr, sl]
            for jr in range(CH):
                pltpu.sync_copy(ones_v, cnt_sh.at[seg_v.at[jr]], add=True)

        plsc.subcore_barrier()
        pltpu.sync_copy(cnt_sh.at[pl.ds(s * SEG_T, SEG_T)], zer_v)
        pltpu.sync_copy(zer_v, out_hbm.at[pl.ds(c * SEGP + s * SEG_T, SEG_T)])

    return sc_cnt


def _make_sc_wgt(er_rows, rows_per_w):
    n_chunks = rows_per_w // CH

    @functools.partial(
        pl.kernel,
        out_type=(jax.ShapeDtypeStruct((er_rows, 128), jnp.float32),
                  jax.ShapeDtypeStruct((er_rows, 128), jnp.int32)),
        mesh=_mesh(),
        compiler_params=pltpu.CompilerParams(needs_layout_passes=False),
        scratch_types=[
            pltpu.VMEM((CH, 128), jnp.int32),
            pltpu.VMEM((CH, 128), jnp.int32),
            pltpu.VMEM((CH, 128), jnp.int32),
            pltpu.VMEM((CH, 128), jnp.float32),
            pltpu.VMEM((CH, 128), jnp.int32),
            pltpu.VMEM((SEGP,), jnp.float32),
        ],
    )
    def sc_wgt(src_hbm, dst_hbm, et_hbm, inv_hbm, w_hbm, g_hbm,
               src_v, dst_v, et_v, w_v, g_v, inv_v):
        c = lax.axis_index("c")
        s = lax.axis_index("s")
        wid = c * NS + s

        pltpu.sync_copy(inv_hbm, inv_v)
        row0 = wid * rows_per_w

        @pl.loop(0, n_chunks)
        def _chunk(k):
            base = row0 + k * CH
            pltpu.sync_copy(src_hbm.at[pl.ds(base, CH)], src_v)
            pltpu.sync_copy(dst_hbm.at[pl.ds(base, CH)], dst_v)
            pltpu.sync_copy(et_hbm.at[pl.ds(base, CH)], et_v)
            for jr in range(CH):
                for jj in range(128 // L):
                    sl = pl.ds(jj * L, L)
                    e16 = et_v.at[jr][sl]
                    seg16 = dst_v.at[jr][sl] * R + e16
                    w_v.at[jr][sl] = plsc.load_gather(inv_v, [seg16])
                    g_v.at[jr][sl] = e16 * N + src_v.at[jr][sl]
            pltpu.sync_copy(w_v, w_hbm.at[pl.ds(base, CH)])
            pltpu.sync_copy(g_v, g_hbm.at[pl.ds(base, CH)])

    return sc_wgt


def _make_sc_msg(er_rows, rows_per_w):
    n_chunks = rows_per_w // CH
    acc_t = N_ACC // NS

    def _scale_rows(rows_v, w_v, jr):
        @pl.loop(0, 128 // L)
        def _scale(g):
            jc = g * L
            w16 = w_v[jr, pl.ds(jc, L)]
            r0 = g * L
            for t in range(L):
                wj = w16[t]
                for q in range(128 // L):
                    sl = pl.ds(q * L, L)
                    rows_v[r0 + t, sl] = rows_v[r0 + t, sl] * wj

    @functools.partial(
        pl.kernel,
        out_type=jax.ShapeDtypeStruct((NC * N_ACC, 128), jnp.float32),
        mesh=_mesh(),
        scratch_types=[
            pltpu.VMEM((CH, 128), jnp.int32),
            pltpu.VMEM((CH, 128), jnp.int32),
            pltpu.VMEM((CH, 128), jnp.float32),
            pltpu.VMEM((CH, 128), jnp.int32),
            pltpu.VMEM((CH, 128), jnp.int32),
            pltpu.VMEM((CH, 128), jnp.float32),
            pltpu.VMEM((128, 128), jnp.float32),
            pltpu.VMEM((128, 128), jnp.float32),
            pltpu.VMEM_SHARED((N_ACC, 128), jnp.float32),
            pltpu.SemaphoreType.DMA,
            pltpu.SemaphoreType.DMA,
            pltpu.SemaphoreType.DMA,
            pltpu.SemaphoreType.DMA,
        ],
    )
    def sc_msg(trans_hbm, g_hbm, dst_hbm, w_hbm, out_hbm,
               g_0, d_0, w_0, g_1, d_1, w_1, rows_a, rows_b, acc_sh,
               sem_a, sem_b, isem_0, isem_1):
        c = lax.axis_index("c")
        s = lax.axis_index("s")
        wid = c * NS + s
        bufs = (rows_a, rows_b)
        sems = (sem_a, sem_b)
        gs = (g_0, g_1)
        dss = (d_0, d_1)
        wss = (w_0, w_1)
        isems = (isem_0, isem_1)

        _fill_zeros(rows_a, 128)
        for i in range(4):
            pltpu.sync_copy(rows_a, acc_sh.at[pl.ds(s * acc_t + i * 128, 128)])
        rem0 = acc_t - 4 * 128
        pltpu.sync_copy(rows_a.at[pl.ds(0, rem0)],
                        acc_sh.at[pl.ds(s * acc_t + 4 * 128, rem0)])
        plsc.subcore_barrier()

        row0 = wid * rows_per_w

        def load_idx(k, slot):
            base = row0 + k * CH
            pltpu.async_copy(g_hbm.at[pl.ds(base, CH)], gs[slot], isems[slot])
            pltpu.async_copy(dst_hbm.at[pl.ds(base, CH)], dss[slot], isems[slot])
            pltpu.async_copy(w_hbm.at[pl.ds(base, CH)], wss[slot], isems[slot])

        def drain_idx(slot):
            pltpu.make_async_copy(g_hbm.at[pl.ds(0, CH)], gs[slot],
                                  isems[slot]).wait()
            pltpu.make_async_copy(dst_hbm.at[pl.ds(0, CH)], dss[slot],
                                  isems[slot]).wait()
            pltpu.make_async_copy(w_hbm.at[pl.ds(0, CH)], wss[slot],
                                  isems[slot]).wait()

        def process(slot):
            g_v, dst_v, w_v = gs[slot], dss[slot], wss[slot]
            cps = [None] * CH
            cps[0] = pltpu.async_copy(trans_hbm.at[g_v.at[0]], bufs[0], sems[0])
            for jr in range(CH):
                sl = jr % 2
                if jr + 1 < CH:
                    cps[jr + 1] = pltpu.async_copy(
                        trans_hbm.at[g_v.at[jr + 1]], bufs[1 - sl], sems[1 - sl])
                cps[jr].wait()
                _scale_rows(bufs[sl], w_v, jr)
                pltpu.sync_copy(bufs[sl], acc_sh.at[dst_v.at[jr]], add=True)

        load_idx(0, 0)

        @pl.loop(0, n_chunks // 2)
        def _pair(kk):
            k0 = kk * 2
            load_idx(k0 + 1, 1)
            drain_idx(0)
            process(0)

            @pl.when(kk < n_chunks // 2 - 1)
            def _prefetch():
                load_idx(k0 + 2, 0)

            drain_idx(1)
            process(1)

        plsc.subcore_barrier()
        for i in range(4):
            pltpu.sync_copy(acc_sh.at[pl.ds(s * acc_t + i * 128, 128)], rows_a)
            pltpu.sync_copy(rows_a,
                            out_hbm.at[pl.ds(c * N_ACC + s * acc_t + i * 128, 128)])
        rem = acc_t - 4 * 128
        pltpu.sync_copy(acc_sh.at[pl.ds(s * acc_t + 4 * 128, rem)],
                        rows_a.at[pl.ds(0, rem)])
        pltpu.sync_copy(rows_a.at[pl.ds(0, rem)],
                        out_hbm.at[pl.ds(c * N_ACC + s * acc_t + 4 * 128, rem)])

    return sc_msg


def _tc_inv(cnt):
    rows = SEGP // 128
    cnt3 = cnt.reshape(NC, rows, 128)

    def body(c_ref, o_ref):
        tot = c_ref[0] + c_ref[1]
        o_ref[...] = 1.0 / jnp.maximum(tot, 1.0)

    out = pl.pallas_call(
        body,
        out_shape=jax.ShapeDtypeStruct((rows, 128), jnp.float32),
    )(cnt3)
    return out.reshape(SEGP)


def _tc_trans(h, W, root, b):
    blk = 1000
    grid = (N // blk,)

    def body(h_ref, w_ref, r_ref, b_ref, t_ref, hr_ref):
        hh = h_ref[...]
        for r in range(R):
            t_ref[r] = jnp.dot(hh, w_ref[r], preferred_element_type=jnp.float32)
        hr_ref[...] = jnp.dot(hh, r_ref[...],
                              preferred_element_type=jnp.float32) + b_ref[...]

    return pl.pallas_call(
        body,
        grid=grid,
        in_specs=[
            pl.BlockSpec((blk, H), lambda i: (i, 0)),
            pl.BlockSpec((R, H, H), lambda i: (0, 0, 0)),
            pl.BlockSpec((H, H), lambda i: (0, 0)),
            pl.BlockSpec((1, H), lambda i: (0, 0)),
        ],
        out_specs=[
            pl.BlockSpec((R, blk, H), lambda i: (0, i, 0)),
            pl.BlockSpec((blk, H), lambda i: (i, 0)),
        ],
        out_shape=[
            jax.ShapeDtypeStruct((R, N, H), jnp.float32),
            jax.ShapeDtypeStruct((N, H), jnp.float32),
        ],
    )(h, W, root, b)


def _tc_combine(m0, m1, hroot):
    blk = 1000
    grid = (N // blk,)

    def body(a_ref, b_ref, c_ref, o_ref):
        o_ref[...] = jnp.maximum(a_ref[...] + b_ref[...] + c_ref[...], 0.0)

    return pl.pallas_call(
        body,
        grid=grid,
        in_specs=[pl.BlockSpec((blk, H), lambda i: (i, 0))] * 3,
        out_specs=pl.BlockSpec((blk, H), lambda i: (i, 0)),
        out_shape=jax.ShapeDtypeStruct((N, H), jnp.float32),
    )(m0, m1, hroot)


def _tc_set2set(h, batch2d, W_ih, W_hh, b_ih, b_hh):
    def body(h_ref, bt_ref, wih_ref, whh_ref, bih_ref, bhh_ref, out_ref):
        hh = h_ref[...]
        bt = bt_ref[...]
        iota = lax.broadcasted_iota(jnp.int32, (1, NB), 1)
        onehot = (bt == iota).astype(jnp.float32)

        wih = wih_ref[...]
        whh = whh_ref[...]
        bih = bih_ref[...]
        bhh = bhh_ref[...]

        q_star = jnp.zeros((NB, 2 * H), jnp.float32)
        hx = jnp.zeros((NB, H), jnp.float32)
        cx = jnp.zeros((NB, H), jnp.float32)

        for _ in range(STEPS):
            gates = (lax.dot_general(q_star, wih, (((1,), (1,)), ((), ())),
                                     preferred_element_type=jnp.float32)
                     + bih
                     + lax.dot_general(hx, whh, (((1,), (1,)), ((), ())),
                                       preferred_element_type=jnp.float32)
                     + bhh)
            gi = jax.nn.sigmoid(gates[:, 0 * H:1 * H])
            gf = jax.nn.sigmoid(gates[:, 1 * H:2 * H])
            gg = jnp.tanh(gates[:, 2 * H:3 * H])
            go = jax.nn.sigmoid(gates[:, 3 * H:4 * H])
            cx = gf * cx + gi * gg
            hx = go * jnp.tanh(cx)
            q = hx

            qb = jnp.dot(onehot, q, preferred_element_type=jnp.float32)
            e = jnp.sum(hh * qb, axis=1, keepdims=True)
            em = jnp.where(onehot > 0.0, e, -1e30)
            m = jnp.max(em, axis=0, keepdims=True)
            m = jnp.where(m > -9e29, m, 0.0)
            mb = jnp.sum(onehot * m, axis=1, keepdims=True)
            ex = jnp.exp(e - mb)
            den = jnp.sum(onehot * ex, axis=0, keepdims=True)
            denb = jnp.sum(onehot * den, axis=1, keepdims=True)
            a = ex / (denb + 1e-16)
            ma = onehot * a
            r = lax.dot_general(ma, hh, (((0,), (0,)), ((), ())),
                                preferred_element_type=jnp.float32)
            q_star = jnp.concatenate([q, r], axis=-1)

        out_ref[...] = q_star

    return pl.pallas_call(
        body,
        out_shape=jax.ShapeDtypeStruct((NB, 2 * H), jnp.float32),
    )(h, batch2d, W_ih, W_hh, b_ih, b_hh)


def kernel(x, edge_index, edge_type, batch, W0, root0, b0, W1, root1, b1,
           W2, root2, b2, W_ih, W_hh, b_ih, b_hh):
    E = edge_index.shape[1]

    rows_per_w = _cdiv(_cdiv(E, 128), NW * CH) * CH
    er_rows = NW * rows_per_w
    ep = er_rows * 128
    pad = ep - E

    src = edge_index[0]
    dst = edge_index[1]
    src_p = jnp.concatenate([src, jnp.zeros((pad,), jnp.int32)]).reshape(er_rows, 128)
    dst_p = jnp.concatenate([dst, jnp.full((pad,), PAD_DST, jnp.int32)]).reshape(er_rows, 128)
    et_p = jnp.concatenate([edge_type, jnp.zeros((pad,), jnp.int32)]).reshape(er_rows, 128)

    cnt = _make_sc_cnt(er_rows, rows_per_w)(dst_p, et_p)
    inv = _tc_inv(cnt.reshape(NC, SEGP))
    wgt, gidx = _make_sc_wgt(er_rows, rows_per_w)(src_p, dst_p, et_p, inv)

    sc_msg = _make_sc_msg(er_rows, rows_per_w)

    h = x
    for (W, root, b) in ((W0, root0, b0), (W1, root1, b1), (W2, root2, b2)):
        trans, hroot = _tc_trans(h, W, root, b.reshape(1, H))
        msg = sc_msg(trans.reshape(R * N, H), gidx, dst_p, wgt)
        msg = msg.reshape(NC, N_ACC, H)
        h = _tc_combine(msg[0, :N], msg[1, :N], hroot)

    emb = _tc_set2set(h, batch.reshape(N, 1), W_ih, W_hh,
                      b_ih.reshape(1, 4 * H), b_hh.reshape(1, 4 * H))
    return (h, emb)
